# baseline (speedup 1.0000x reference)
import functools

import jax
import jax.numpy as jnp
from jax import lax
from jax.experimental import pallas as pl
from jax.experimental.pallas import tpu as pltpu
from jax.experimental.pallas import tpu_sc as plsc

CANVAS = 200
HALF = 15
B, E, H, W = 64, 256, 31, 31
CPX = CANVAS * CANVAS
NW = 32
BATCH_PER_W = B // NW
CH = 4
NWAVE = 8
EC = 128
ST = 33


def _sc_scatter(t6, x, y):
    mesh = plsc.VectorSubcoreMesh(core_axis_name="c", subcore_axis_name="s")

    @functools.partial(
        pl.kernel,
        mesh=mesh,
        out_type=jax.ShapeDtypeStruct((B, CPX), jnp.float32),
        scratch_types=[
            pltpu.VMEM((CPX,), jnp.float32),
            pltpu.VMEM((CH, 32, EC), jnp.float32),
            pltpu.VMEM((CH, 32, EC), jnp.float32),
            pltpu.VMEM((CH, 32, EC), jnp.float32),
            pltpu.VMEM((CH, 32, EC), jnp.float32),
            pltpu.VMEM((CH * EC * ST,), jnp.float32),
            pltpu.VMEM((E,), jnp.int32),
            pltpu.VMEM((E,), jnp.int32),
            pltpu.VMEM((E,), jnp.int32),
            pltpu.SemaphoreType.DMA,
            pltpu.SemaphoreType.DMA,
            pltpu.SemaphoreType.DMA,
            pltpu.SemaphoreType.DMA,
        ],
        compiler_params=pltpu.CompilerParams(needs_layout_passes=False),
    )
    def k(t6_hbm, x_hbm, y_hbm, out_hbm,
          canvas, bufA0, bufA1, bufB0, bufB1, bufT, xbuf, ybuf, base,
          semA0, semA1, semB0, semB1):
        wid = lax.axis_index("s") * 2 + lax.axis_index("c")
        iota = lax.iota(jnp.int32, 16)
        iota33 = iota * ST
        mask1 = iota >= 1
        zero16 = jnp.zeros((16,), jnp.int32)
        zerosf = jnp.zeros((16,), jnp.float32)
        bufs = ((bufA0, bufA1), (bufB0, bufB1))
        sems = ((semA0, semA1), (semB0, semB1))

        def wave_copy(b, h0, nh, half, par):
            bi = b // 8
            br = b - bi * 8
            src = t6_hbm.at[pl.ds(h0, nh), :, bi, half, br, :]
            dst = bufs[half][par].at[pl.ds(0, nh), pl.ds(0, W), :]
            return pltpu.make_async_copy(src, dst, sems[half][par])

        def batch_body(t, carry):
            b = wid * BATCH_PER_W + t

            cps = [wave_copy(b, 0, CH, 0, 0), wave_copy(b, 0, CH, 1, 0)]
            cps[0].start()
            cps[1].start()

            def zbody(i, carry):
                canvas[pl.ds(i * 16, 16)] = zerosf
                return carry
            lax.fori_loop(0, CPX // 16, zbody, 0)

            pltpu.sync_copy(x_hbm.at[pl.ds(b * E, E)], xbuf)
            pltpu.sync_copy(y_hbm.at[pl.ds(b * E, E)], ybuf)

            def bbody(i, carry):
                xv = xbuf[pl.ds(i * 16, 16)]
                yv = ybuf[pl.ds(i * 16, 16)]
                base[pl.ds(i * 16, 16)] = (xv - HALF) * CANVAS + (yv - HALF)
                return carry
            lax.fori_loop(0, E // 16, bbody, 0)

            for wv in range(NWAVE):
                h0 = wv * CH
                nh = min(CH, H - h0)
                par = wv % 2
                if wv + 1 < NWAVE:
                    n0 = (wv + 1) * CH
                    nnh = min(CH, H - n0)
                    nxt = [wave_copy(b, n0, nnh, 0, 1 - par),
                           wave_copy(b, n0, nnh, 1, 1 - par)]
                    nxt[0].start()
                    nxt[1].start()
                cps[0].wait()
                cps[1].wait()

                for half in range(2):
                    buf = bufs[half][par]

                    @plsc.parallel_loop(0, W, unroll=2)
                    def tw(w, buf=buf, nh=nh):
                        for hh in range(nh):
                            for ecb in range(EC // 16):
                                v = buf[hh, w, pl.ds(ecb * 16, 16)]
                                tidx = iota33 + (
                                    (hh * EC + ecb * 16) * ST + w)
                                plsc.store_scatter(bufT, [tidx], v)

                    @plsc.parallel_loop(0, EC)
                    def ebody(ec, half=half, nh=nh, h0=h0):
                        e = half * EC + ec
                        bs = plsc.load_gather(base, [zero16 + e])
                        b0 = bs + iota
                        ti = ec * ST
                        for hh in range(nh):
                            t0 = ti + hh * (EC * ST)
                            v0 = bufT[pl.ds(t0, 16)]
                            v1 = bufT[pl.ds(t0 + (W - 16), 16)]
                            sv0 = b0 + (h0 + hh) * CANVAS
                            plsc.addupdate_scatter(canvas, [sv0], v0)
                            plsc.addupdate_scatter(canvas,
                                                   [sv0 + (W - 16)], v1,
                                                   mask=mask1)

                if wv + 1 < NWAVE:
                    cps = nxt

            pltpu.sync_copy(canvas, out_hbm.at[b])
            return carry

        lax.fori_loop(0, BATCH_PER_W, batch_body, 0)

    return k(t6, x, y)


def kernel(images4D, xyz):
    t6 = images4D.reshape(8, 8, 2, 128, H, W).transpose(4, 5, 0, 2, 1, 3)
    x = xyz[:, :, 0].reshape(-1).astype(jnp.int32)
    y = xyz[:, :, 1].reshape(-1).astype(jnp.int32)
    out = _sc_scatter(t6, x, y)
    return out.reshape(B, 1, CANVAS, CANVAS)


# speedup vs baseline: 1.0401x; 1.0401x over previous
import functools

import jax
import jax.numpy as jnp
from jax import lax
from jax.experimental import pallas as pl
from jax.experimental.pallas import tpu as pltpu
from jax.experimental.pallas import tpu_sc as plsc

CANVAS = 200
HALF = 15
B, E, H, W = 64, 256, 31, 31
CPX = CANVAS * CANVAS
NW = 32
BATCH_PER_W = B // NW
CH = 4
NWAVE = 8
EC = 128
ST = 33


def _sc_scatter(t6, x, y):
    mesh = plsc.VectorSubcoreMesh(core_axis_name="c", subcore_axis_name="s")

    @functools.partial(
        pl.kernel,
        mesh=mesh,
        out_type=jax.ShapeDtypeStruct((B, CPX), jnp.float32),
        scratch_types=[
            pltpu.VMEM((CPX,), jnp.float32),
            pltpu.VMEM((CH, 32, EC), jnp.float32),
            pltpu.VMEM((CH, 32, EC), jnp.float32),
            pltpu.VMEM((CH, 32, EC), jnp.float32),
            pltpu.VMEM((CH, 32, EC), jnp.float32),
            pltpu.VMEM((CH * EC * ST,), jnp.float32),
            pltpu.VMEM((E,), jnp.int32),
            pltpu.VMEM((E,), jnp.int32),
            pltpu.VMEM((E,), jnp.int32),
            pltpu.SemaphoreType.DMA,
            pltpu.SemaphoreType.DMA,
            pltpu.SemaphoreType.DMA,
            pltpu.SemaphoreType.DMA,
        ],
        compiler_params=pltpu.CompilerParams(needs_layout_passes=False),
    )
    def k(t6_hbm, x_hbm, y_hbm, out_hbm,
          canvas, bufA0, bufA1, bufB0, bufB1, bufT, xbuf, ybuf, base,
          semA0, semA1, semB0, semB1):
        wid = lax.axis_index("s") * 2 + lax.axis_index("c")
        iota = lax.iota(jnp.int32, 16)
        iota33 = iota * ST
        mask1 = iota >= 1
        zero16 = jnp.zeros((16,), jnp.int32)
        zerosf = jnp.zeros((16,), jnp.float32)
        bufs = ((bufA0, bufA1), (bufB0, bufB1))
        sems = ((semA0, semA1), (semB0, semB1))

        def wave_copy(b, h0, nh, half, par):
            bi = b // 8
            br = b - bi * 8
            src = t6_hbm.at[pl.ds(h0, nh), :, bi, half, br, :]
            dst = bufs[half][par].at[pl.ds(0, nh), pl.ds(0, W), :]
            return pltpu.make_async_copy(src, dst, sems[half][par])

        def batch_body(t, carry):
            b = wid * BATCH_PER_W + t

            cps = [wave_copy(b, 0, CH, 0, 0), wave_copy(b, 0, CH, 1, 0)]
            cps[0].start()
            cps[1].start()

            def zbody(i, carry):
                canvas[pl.ds(i * 16, 16)] = zerosf
                return carry
            lax.fori_loop(0, CPX // 16, zbody, 0)

            pltpu.sync_copy(x_hbm.at[pl.ds(b * E, E)], xbuf)
            pltpu.sync_copy(y_hbm.at[pl.ds(b * E, E)], ybuf)

            def bbody(i, carry):
                xv = xbuf[pl.ds(i * 16, 16)]
                yv = ybuf[pl.ds(i * 16, 16)]
                base[pl.ds(i * 16, 16)] = (xv - HALF) * CANVAS + (yv - HALF)
                return carry
            lax.fori_loop(0, E // 16, bbody, 0)

            for wv in range(NWAVE):
                h0 = wv * CH
                nh = min(CH, H - h0)
                par = wv % 2
                if wv + 1 < NWAVE:
                    n0 = (wv + 1) * CH
                    nnh = min(CH, H - n0)
                    nxt = [wave_copy(b, n0, nnh, 0, 1 - par),
                           wave_copy(b, n0, nnh, 1, 1 - par)]
                    nxt[0].start()
                    nxt[1].start()
                cps[0].wait()
                cps[1].wait()

                for half in range(2):
                    buf = bufs[half][par]

                    @plsc.parallel_loop(0, W)
                    def tw(w, buf=buf, nh=nh):
                        for hh in range(nh):
                            for ecb in range(EC // 16):
                                v = buf[hh, w, pl.ds(ecb * 16, 16)]
                                tidx = iota33 + (
                                    (hh * EC + ecb * 16) * ST + w)
                                plsc.store_scatter(bufT, [tidx], v)

                    @plsc.parallel_loop(0, EC)
                    def ebody(ec, half=half, nh=nh, h0=h0):
                        e = half * EC + ec
                        bs = plsc.load_gather(base, [zero16 + e])
                        b0 = bs + iota
                        ti = ec * ST
                        for hh in range(nh):
                            t0 = ti + hh * (EC * ST)
                            v0 = bufT[pl.ds(t0, 16)]
                            v1 = bufT[pl.ds(t0 + (W - 16), 16)]
                            sv0 = b0 + (h0 + hh) * CANVAS
                            plsc.addupdate_scatter(canvas, [sv0], v0)
                            plsc.addupdate_scatter(canvas,
                                                   [sv0 + (W - 16)], v1,
                                                   mask=mask1)

                if wv + 1 < NWAVE:
                    cps = nxt

            pltpu.sync_copy(canvas, out_hbm.at[b])
            return carry

        lax.fori_loop(0, BATCH_PER_W, batch_body, 0)

    return k(t6, x, y)


def kernel(images4D, xyz):
    t6 = images4D.reshape(8, 8, 2, 128, H, W).transpose(4, 5, 0, 2, 1, 3)
    x = xyz[:, :, 0].reshape(-1).astype(jnp.int32)
    y = xyz[:, :, 1].reshape(-1).astype(jnp.int32)
    out = _sc_scatter(t6, x, y)
    return out.reshape(B, 1, CANVAS, CANVAS)
